# Initial kernel scaffold; baseline (speedup 1.0000x reference)
import jax
import jax.numpy as jnp
from jax.experimental import pallas as pl


def kernel(score_map, geo_map):
    raise NotImplementedError("write your pallas kernel here")


# speedup vs baseline: 9.5347x; 9.5347x over previous
import dataclasses
import functools

import jax
import jax.numpy as jnp
from jax import lax
from jax.experimental import pallas as pl
from jax.experimental.pallas import tpu as pltpu
from jax.experimental.pallas import tpu_sc as plsc

H = 320
W = 320
N = H * W
K = 2000
B08 = 0x3F4CCCCD
NMS_T = 0.2

NSUB = N // 32
PC = 2304
G = 2 * PC
S = 2048
SENT = N
NPAD = N + 16


def _thresh_body(score_ref, thr_ref):
    bits = lax.bitcast_convert_type(score_ref[...], jnp.int32)

    def bs_body(_, lohi):
        lo, hi = lohi
        mid = (lo + hi) // 2
        p = jnp.sum((bits >= mid).astype(jnp.int32)) >= K
        return jnp.where(p, mid, lo), jnp.where(p, hi, mid)

    lo, _ = lax.fori_loop(0, 30, bs_body,
                          (jnp.int32(0), jnp.int32(1 << 30)))
    t = lo
    cnt_gt = jnp.sum((bits > t).astype(jnp.int32))
    needed = K - cnt_gt
    r8 = lax.broadcasted_iota(jnp.int32, (800, 128), 0)
    l8 = lax.broadcasted_iota(jnp.int32, (800, 128), 1)
    pos = r8 * 128 + l8
    eqt = bits == t

    def ts_body(_, lohi):
        lo2, hi2 = lohi
        mid = (lo2 + hi2) // 2
        q = jnp.sum((eqt & (pos < mid)).astype(jnp.int32)) >= needed
        return jnp.where(q, lo2, mid), jnp.where(q, mid, hi2)

    _, hi2 = lax.fori_loop(0, 17, ts_body,
                           (jnp.int32(0), jnp.int32(1 << 17)))
    use_tie = t > B08
    t_eff = jnp.maximum(t, B08)
    icut = jnp.where(use_tie, hi2, 0)
    thr_ref[0:1, :] = jnp.full((1, 128), t_eff, jnp.int32)
    thr_ref[1:2, :] = jnp.full((1, 128), icut, jnp.int32)


def _thresholds(score2):
    return pl.pallas_call(
        _thresh_body,
        out_shape=jax.ShapeDtypeStruct((2, 128), jnp.int32),
    )(score2)


LPC = 2048


def _sc_cp():
    cp = pltpu.CompilerParams()
    if "needs_layout_passes" in pltpu.CompilerParams.__dataclass_fields__:
        cp = dataclasses.replace(cp, needs_layout_passes=False)
    return cp


def _sc1_body(score_hbm, thr_hbm, lists_hbm, wcnt_hbm,
              sc_score, idxbuf, tthr, ticut, stage, sem):
    c = lax.axis_index("c")
    s = lax.axis_index("s")
    w = c * 16 + s
    base = c * (N // 2) + s * NSUB

    pltpu.sync_copy(score_hbm.at[pl.ds(base, NSUB)], sc_score)
    pltpu.sync_copy(thr_hbm.at[pl.ds(0, 16)], tthr)
    pltpu.sync_copy(thr_hbm.at[pl.ds(128, 16)], ticut)
    tvec = tthr[...]
    icvec = ticut[...]

    sent = jnp.full((16,), SENT, jnp.int32)

    @pl.loop(0, NSUB + 32, step=16)
    def _(j):
        idxbuf[pl.ds(j, 16)] = sent

    def chunk_body(ch, off):
        v = sc_score[pl.ds(ch * 16, 16)]
        bits = plsc.bitcast(v, jnp.int32)
        idxv = base + ch * 16 + lax.iota(jnp.int32, 16)
        m = (bits > tvec) | ((bits == tvec) & (idxv < icvec))
        plsc.store_compressed(idxbuf.at[pl.ds(off, 16)], idxv, mask=m)
        return off + jnp.sum(m.astype(jnp.int32))

    cnt = lax.fori_loop(0, NSUB // 16, chunk_body, jnp.int32(0))
    idxbuf[pl.ds(cnt, 16)] = sent
    cnt = jnp.minimum(cnt, K)

    pltpu.sync_copy(idxbuf.at[pl.ds(0, LPC)],
                    lists_hbm.at[pl.ds(w * LPC, LPC)])
    stage[...] = jnp.full((16,), cnt, jnp.int32)
    pltpu.sync_copy(stage, wcnt_hbm.at[pl.ds(w * 16, 16)])


def _sc_compact(score_flat, thr_flat):
    mesh = plsc.VectorSubcoreMesh(core_axis_name="c", subcore_axis_name="s")
    kern = pl.kernel(
        _sc1_body,
        out_type=[jax.ShapeDtypeStruct((32 * LPC,), jnp.int32),
                  jax.ShapeDtypeStruct((512,), jnp.int32)],
        mesh=mesh,
        scratch_types=[
            pltpu.VMEM((NSUB,), jnp.float32),
            pltpu.VMEM((NSUB + 32,), jnp.int32),
            pltpu.VMEM((16,), jnp.int32),
            pltpu.VMEM((16,), jnp.int32),
            pltpu.VMEM((16,), jnp.int32),
            pltpu.SemaphoreType.DMA,
        ],
        compiler_params=_sc_cp(),
    )
    return kern(score_flat, thr_flat)


def _prefix_body(wcnt_ref, offs_ref, cnts_ref, used_ref):
    offs = []
    cntps = []
    run = [jnp.int32(0), jnp.int32(0)]
    for w in range(32):
        c = w // 16
        cw = wcnt_ref[w, 0]
        cp16 = ((cw + 15) // 16) * 16
        offs.append(jnp.minimum(run[c], PC - cp16))
        cntps.append(cp16)
        run[c] = run[c] + cp16
    for w in range(32):
        c = w // 16
        offs_ref[w:w + 1, :] = jnp.full((1, 16), c * PC + offs[w],
                                        jnp.int32)
        cnts_ref[w:w + 1, :] = jnp.full((1, 16), cntps[w], jnp.int32)
    used_ref[0:1, :] = jnp.full((1, 16), jnp.minimum(run[0], PC), jnp.int32)
    used_ref[1:2, :] = jnp.full((1, 16), jnp.minimum(run[1], PC), jnp.int32)


def _prefix(wcnt):
    return pl.pallas_call(
        _prefix_body,
        out_shape=[jax.ShapeDtypeStruct((32, 16), jnp.int32),
                   jax.ShapeDtypeStruct((32, 16), jnp.int32),
                   jax.ShapeDtypeStruct((2, 16), jnp.int32)],
    )(wcnt)


def _sc2_body(lists_hbm, offs_hbm, cnts_hbm, packed_hbm, rows_hbm,
              idxbuf, obuf, cbuf, gidx, rowbuf, sem):
    c = lax.axis_index("c")
    s = lax.axis_index("s")
    w = c * 16 + s

    pltpu.sync_copy(lists_hbm.at[pl.ds(w * LPC, LPC)], idxbuf)
    pltpu.sync_copy(offs_hbm.at[pl.ds(w * 16, 16)], obuf)
    pltpu.sync_copy(cnts_hbm.at[pl.ds(w * 16, 16)], cbuf)
    g_off = jnp.max(obuf[...])
    nch = jnp.max(cbuf[...]) // 16

    @pl.loop(0, LPC // 16)
    def _(j):
        @pl.when(j < nch)
        def _():
            iv = idxbuf[pl.ds(j * 16, 16)]
            pltpu.async_copy(packed_hbm.at[plsc.Indices(iv)],
                             rowbuf, sem).wait()
            dst = pl.multiple_of(g_off + j * 16, 16)
            pltpu.sync_copy(rowbuf, rows_hbm.at[pl.ds(dst, 16)])


def _sc_gather(lists, offs, cnts, packed):
    mesh = plsc.VectorSubcoreMesh(core_axis_name="c", subcore_axis_name="s")
    kern = pl.kernel(
        _sc2_body,
        out_type=jax.ShapeDtypeStruct((G, 128), jnp.float32),
        mesh=mesh,
        scratch_types=[
            pltpu.VMEM((LPC,), jnp.int32),
            pltpu.VMEM((16,), jnp.int32),
            pltpu.VMEM((16,), jnp.int32),
            pltpu.VMEM((16,), jnp.int32),
            pltpu.VMEM((16, 128), jnp.float32),
            pltpu.SemaphoreType.DMA,
        ],
        compiler_params=_sc_cp(),
    )
    return kern(lists, offs, cnts, packed)


def _nms_body(rows_ref, cnt_ref, out_ref,
              bits_ref, idx_ref, sorted_ref,
              xmin_ref, ymin_ref, xmax_ref, ymax_ref, area_ref, mask_ref):
    f32 = jnp.float32
    i32 = jnp.int32

    rows = rows_ref[:, 0:16]
    rt = jnp.transpose(rows)

    def col(k):
        return rt[k].reshape(36, 128)

    score = col(0)
    u0 = cnt_ref[0, 0]
    u1 = cnt_ref[0, 16]
    r36 = lax.broadcasted_iota(i32, (36, 128), 0)
    l36 = lax.broadcasted_iota(i32, (36, 128), 1)
    pos = r36 * 128 + l36
    vpos = (pos < u0) | ((pos >= PC) & (pos < PC + u1))
    score = jnp.where(vpos, score, 0.0)
    xsf = jnp.where(vpos, col(6), 0.0)
    ysf = jnp.where(vpos, col(7), 0.0)
    bits = lax.bitcast_convert_type(score, i32)
    idxf = ysf * 320.0 + xsf

    bits_ref[...] = bits.reshape(1, G)
    idx_ref[...] = idxf.reshape(1, G)
    pcol = lax.broadcasted_iota(i32, (G, 1), 0)
    vcol = (pcol < u0) | ((pcol >= PC) & (pcol < PC + u1))
    rows = jnp.where(vcol, rows, 0.0)

    bits_c = bits_ref[...]
    idx_c = idx_ref[...]

    def rank_body(rb, ranks):
        bj = jnp.transpose(bits_ref[0:1, pl.ds(rb * 128, 128)])
        ij = jnp.transpose(idx_ref[0:1, pl.ds(rb * 128, 128)])
        gt = bj > bits_c
        tie = (bj == bits_c) & (ij < idx_c)
        return ranks + jnp.sum((gt | tie).astype(i32), axis=0,
                               keepdims=True)

    ranks = lax.fori_loop(0, G // 128, rank_body,
                          jnp.zeros((1, G), i32))
    realc = bits_c != 0

    def perm_body(sb, carry):
        slots = sb * 256 + lax.broadcasted_iota(i32, (256, 1), 0)
        pmat = ((ranks == slots) & realc).astype(f32)
        blk = jax.lax.dot_general(pmat, rows,
                                  (((1,), (0,)), ((), ())),
                                  precision=jax.lax.Precision.HIGHEST,
                                  preferred_element_type=f32)
        sorted_ref[pl.ds(sb * 256, 256), :] = blk
        return carry

    lax.fori_loop(0, S // 256, perm_body, jnp.int32(0))

    srt = jnp.transpose(sorted_ref[...])

    def scol(k):
        return srt[k].reshape(16, 128)

    sscore = scol(0)
    d0, d1, d2, d3 = scol(1), scol(2), scol(3), scol(4)
    ang = scol(5)
    x0 = scol(6) * 4.0
    y0 = scol(7) * 4.0

    wsum = d1 + d3
    hsum = d0 + d2
    c0 = jnp.cos(ang)
    s0 = jnp.sin(ang)
    p4x = c0 * d3 + s0 * (-d2)
    p4y = -s0 * d3 + c0 * (-d2)
    ox = x0 - p4x
    oy = y0 - p4y
    q0x = s0 * (-hsum) + ox
    q0y = c0 * (-hsum) + oy
    q1x = c0 * wsum + s0 * (-hsum) + ox
    q1y = -s0 * wsum + c0 * (-hsum) + oy
    q2x = c0 * wsum + ox
    q2y = -s0 * wsum + oy
    q3x = ox
    q3y = oy
    c1 = jnp.cos(-ang)
    s1 = jnp.sin(-ang)
    r4x = c1 * (-d1) - s1 * (-d2)
    r4y = s1 * (-d1) + c1 * (-d2)
    ox1 = x0 - r4x
    oy1 = y0 - r4y
    r0x = c1 * (-wsum) - s1 * (-hsum) + ox1
    r0y = s1 * (-wsum) + c1 * (-hsum) + oy1
    r1x = -s1 * (-hsum) + ox1
    r1y = c1 * (-hsum) + oy1
    r2x = ox1
    r2y = oy1
    r3x = c1 * (-wsum) + ox1
    r3y = s1 * (-wsum) + oy1

    ge = ang >= 0.0
    xa = jnp.where(ge, q0x, r0x)
    xb = jnp.where(ge, q1x, r1x)
    xc = jnp.where(ge, q2x, r2x)
    xd = jnp.where(ge, q3x, r3x)
    ya = jnp.where(ge, q0y, r0y)
    yb = jnp.where(ge, q1y, r1y)
    yc = jnp.where(ge, q2y, r2y)
    yd = jnp.where(ge, q3y, r3y)
    xmin = jnp.minimum(jnp.minimum(xa, xb), jnp.minimum(xc, xd))
    xmax = jnp.maximum(jnp.maximum(xa, xb), jnp.maximum(xc, xd))
    ymin = jnp.minimum(jnp.minimum(ya, yb), jnp.minimum(yc, yd))
    ymax = jnp.maximum(jnp.maximum(ya, yb), jnp.maximum(yc, yd))
    area = jnp.maximum(xmax - xmin, 0.0) * jnp.maximum(ymax - ymin, 0.0)

    xmin_ref[...] = xmin
    ymin_ref[...] = ymin
    xmax_ref[...] = xmax
    ymax_ref[...] = ymax
    area_ref[...] = area

    l128c = lax.broadcasted_iota(i32, (1, 128), 1)
    r128 = lax.broadcasted_iota(i32, (128, 1), 0)

    def mrow_body(rb, carry):
        xnr = jnp.transpose(xmin_ref[pl.ds(rb, 1), :])
        ynr = jnp.transpose(ymin_ref[pl.ds(rb, 1), :])
        xxr = jnp.transpose(xmax_ref[pl.ds(rb, 1), :])
        yxr = jnp.transpose(ymax_ref[pl.ds(rb, 1), :])
        ar = jnp.transpose(area_ref[pl.ds(rb, 1), :])
        rslot = rb * 128 + r128
        tiles = []
        for g in range(16):
            xnc = xmin_ref[g:g + 1, :]
            ync = ymin_ref[g:g + 1, :]
            xxc = xmax_ref[g:g + 1, :]
            yxc = ymax_ref[g:g + 1, :]
            ac = area_ref[g:g + 1, :]
            ix1 = jnp.maximum(xnr, xnc)
            iy1 = jnp.maximum(ynr, ync)
            ix2 = jnp.minimum(xxr, xxc)
            iy2 = jnp.minimum(yxr, yxc)
            inter = (jnp.maximum(ix2 - ix1, 0.0)
                     * jnp.maximum(iy2 - iy1, 0.0))
            union = ar + ac - inter
            cslot = g * 128 + l128c
            m = ((inter > NMS_T * union) & (cslot > rslot)).astype(f32)
            tiles.append(m)
        mask_ref[pl.ds(rb * 128, 128)] = jnp.stack(tiles, axis=1)
        return carry

    lax.fori_loop(0, S // 128, mrow_body, jnp.int32(0))

    r16 = lax.broadcasted_iota(i32, (16, 128), 0)
    l16 = lax.broadcasted_iota(i32, (16, 128), 1)
    slotpos = r16 * 128 + l16
    valid = ((sscore > 0.8) & (slotpos < K)).astype(f32)

    def nms_body(i, keep):
        row = mask_ref[i]
        ki = jnp.max(jnp.where(slotpos == i, keep, 0.0))
        return keep * (1.0 - ki * row)

    keep = lax.fori_loop(0, K, nms_body, valid)

    b0 = xmin * keep
    b1 = ymin * keep
    b2 = xmax * keep
    b3 = ymax * keep
    sc = sscore * keep
    five = jnp.stack([b0, b1, b2, b3, sc], axis=0)
    out_ref[...] = jnp.transpose(five.reshape(5, S))[:K, :]


def _nms(rows, cpad):
    return pl.pallas_call(
        _nms_body,
        out_shape=jax.ShapeDtypeStruct((K, 5), jnp.float32),
        scratch_shapes=[
            pltpu.VMEM((1, G), jnp.int32),
            pltpu.VMEM((1, G), jnp.float32),
            pltpu.VMEM((S, 16), jnp.float32),
            pltpu.VMEM((16, 128), jnp.float32),
            pltpu.VMEM((16, 128), jnp.float32),
            pltpu.VMEM((16, 128), jnp.float32),
            pltpu.VMEM((16, 128), jnp.float32),
            pltpu.VMEM((16, 128), jnp.float32),
            pltpu.VMEM((S, 16, 128), jnp.float32),
        ],
    )(rows, cpad)


def kernel(score_map, geo_map):
    score_flat = score_map.reshape(-1)
    thr = _thresholds(score_flat.reshape(800, 128)).reshape(-1)
    idx = jnp.arange(N, dtype=jnp.int32)
    xs = (idx % W).astype(jnp.float32)
    ys = (idx // W).astype(jnp.float32)
    packed = jnp.concatenate(
        [score_flat[:, None], geo_map.reshape(N, 5),
         xs[:, None], ys[:, None]], axis=1)
    packed = jnp.pad(packed, ((0, NPAD - N), (0, 120)))
    lists, wcnt = _sc_compact(score_flat, thr)
    offs, cnts, used = _prefix(wcnt.reshape(32, 16))
    rows = _sc_gather(lists, offs.reshape(512), cnts.reshape(512), packed)
    cpad = jnp.pad(used.reshape(32), (0, 96)).reshape(1, 128)
    return _nms(rows, cpad)


# speedup vs baseline: 9.8342x; 1.0314x over previous
import dataclasses
import functools

import jax
import jax.numpy as jnp
from jax import lax
from jax.experimental import pallas as pl
from jax.experimental.pallas import tpu as pltpu
from jax.experimental.pallas import tpu_sc as plsc

H = 320
W = 320
N = H * W
K = 2000
B08 = 0x3F4CCCCD
NMS_T = 0.2

NSUB = N // 32
PC = 2304
G = 2 * PC
S = 2048
SENT = N
NPAD = N + 16


def _thresh_body(score_ref, thr_ref):
    bits = lax.bitcast_convert_type(score_ref[...], jnp.int32)

    def bs_body(_, lohi):
        lo, hi = lohi
        mid = (lo + hi) // 2
        p = jnp.sum((bits >= mid).astype(jnp.int32)) >= K
        return jnp.where(p, mid, lo), jnp.where(p, hi, mid)

    lo, _ = lax.fori_loop(0, 30, bs_body,
                          (jnp.int32(0), jnp.int32(1 << 30)))
    t = lo
    cnt_gt = jnp.sum((bits > t).astype(jnp.int32))
    needed = K - cnt_gt
    r8 = lax.broadcasted_iota(jnp.int32, (800, 128), 0)
    l8 = lax.broadcasted_iota(jnp.int32, (800, 128), 1)
    pos = r8 * 128 + l8
    eqt = bits == t

    def ts_body(_, lohi):
        lo2, hi2 = lohi
        mid = (lo2 + hi2) // 2
        q = jnp.sum((eqt & (pos < mid)).astype(jnp.int32)) >= needed
        return jnp.where(q, lo2, mid), jnp.where(q, mid, hi2)

    _, hi2 = lax.fori_loop(0, 17, ts_body,
                           (jnp.int32(0), jnp.int32(1 << 17)))
    use_tie = t > B08
    t_eff = jnp.maximum(t, B08)
    icut = jnp.where(use_tie, hi2, 0)
    thr_ref[0:1, :] = jnp.full((1, 128), t_eff, jnp.int32)
    thr_ref[1:2, :] = jnp.full((1, 128), icut, jnp.int32)


def _thresholds(score2):
    return pl.pallas_call(
        _thresh_body,
        out_shape=jax.ShapeDtypeStruct((2, 128), jnp.int32),
    )(score2)


LPC = 2048


def _sc_cp():
    cp = pltpu.CompilerParams()
    if "needs_layout_passes" in pltpu.CompilerParams.__dataclass_fields__:
        cp = dataclasses.replace(cp, needs_layout_passes=False)
    return cp


def _sc1_body(score_hbm, thr_hbm, lists_hbm, wcnt_hbm,
              sc_score, idxbuf, tthr, ticut, stage, sem):
    c = lax.axis_index("c")
    s = lax.axis_index("s")
    w = c * 16 + s
    base = c * (N // 2) + s * NSUB

    pltpu.sync_copy(score_hbm.at[pl.ds(base, NSUB)], sc_score)
    pltpu.sync_copy(thr_hbm.at[pl.ds(0, 16)], tthr)
    pltpu.sync_copy(thr_hbm.at[pl.ds(128, 16)], ticut)
    tvec = tthr[...]
    icvec = ticut[...]

    sent = jnp.full((16,), SENT, jnp.int32)

    @pl.loop(0, NSUB + 32, step=16)
    def _(j):
        idxbuf[pl.ds(j, 16)] = sent

    def chunk_body(ch, off):
        v = sc_score[pl.ds(ch * 16, 16)]
        bits = plsc.bitcast(v, jnp.int32)
        idxv = base + ch * 16 + lax.iota(jnp.int32, 16)
        m = (bits > tvec) | ((bits == tvec) & (idxv < icvec))
        plsc.store_compressed(idxbuf.at[pl.ds(off, 16)], idxv, mask=m)
        return off + jnp.sum(m.astype(jnp.int32))

    cnt = lax.fori_loop(0, NSUB // 16, chunk_body, jnp.int32(0))
    idxbuf[pl.ds(cnt, 16)] = sent
    cnt = jnp.minimum(cnt, K)

    pltpu.sync_copy(idxbuf.at[pl.ds(0, LPC)],
                    lists_hbm.at[pl.ds(w * LPC, LPC)])
    stage[...] = jnp.full((16,), cnt, jnp.int32)
    pltpu.sync_copy(stage, wcnt_hbm.at[pl.ds(w * 16, 16)])


def _sc_compact(score_flat, thr_flat):
    mesh = plsc.VectorSubcoreMesh(core_axis_name="c", subcore_axis_name="s")
    kern = pl.kernel(
        _sc1_body,
        out_type=[jax.ShapeDtypeStruct((32 * LPC,), jnp.int32),
                  jax.ShapeDtypeStruct((512,), jnp.int32)],
        mesh=mesh,
        scratch_types=[
            pltpu.VMEM((NSUB,), jnp.float32),
            pltpu.VMEM((NSUB + 32,), jnp.int32),
            pltpu.VMEM((16,), jnp.int32),
            pltpu.VMEM((16,), jnp.int32),
            pltpu.VMEM((16,), jnp.int32),
            pltpu.SemaphoreType.DMA,
        ],
        compiler_params=_sc_cp(),
    )
    return kern(score_flat, thr_flat)


def _prefix_body(wcnt_ref, offs_ref, cnts_ref, used_ref):
    offs = []
    cntps = []
    run = [jnp.int32(0), jnp.int32(0)]
    for w in range(32):
        c = w // 16
        cw = wcnt_ref[w, 0]
        cp16 = ((cw + 15) // 16) * 16
        offs.append(jnp.minimum(run[c], PC - cp16))
        cntps.append(cp16)
        run[c] = run[c] + cp16
    for w in range(32):
        c = w // 16
        offs_ref[w:w + 1, :] = jnp.full((1, 16), c * PC + offs[w],
                                        jnp.int32)
        cnts_ref[w:w + 1, :] = jnp.full((1, 16), cntps[w], jnp.int32)
    used_ref[0:1, :] = jnp.full((1, 16), jnp.minimum(run[0], PC), jnp.int32)
    used_ref[1:2, :] = jnp.full((1, 16), jnp.minimum(run[1], PC), jnp.int32)


def _prefix(wcnt):
    return pl.pallas_call(
        _prefix_body,
        out_shape=[jax.ShapeDtypeStruct((32, 16), jnp.int32),
                   jax.ShapeDtypeStruct((32, 16), jnp.int32),
                   jax.ShapeDtypeStruct((2, 16), jnp.int32)],
    )(wcnt)


def _sc2_body(lists_hbm, offs_hbm, cnts_hbm, packed_hbm, rows_hbm,
              idxbuf, obuf, cbuf, gidx, rowbuf, sem):
    c = lax.axis_index("c")
    s = lax.axis_index("s")
    w = c * 16 + s

    pltpu.sync_copy(lists_hbm.at[pl.ds(w * LPC, LPC)], idxbuf)
    pltpu.sync_copy(offs_hbm.at[pl.ds(w * 16, 16)], obuf)
    pltpu.sync_copy(cnts_hbm.at[pl.ds(w * 16, 16)], cbuf)
    g_off = jnp.max(obuf[...])
    nch = jnp.max(cbuf[...]) // 16

    @pl.loop(0, LPC // 16)
    def _(j):
        @pl.when(j < nch)
        def _():
            iv = idxbuf[pl.ds(j * 16, 16)]
            pltpu.async_copy(packed_hbm.at[plsc.Indices(iv)],
                             rowbuf, sem).wait()
            dst = pl.multiple_of(g_off + j * 16, 16)
            pltpu.sync_copy(rowbuf, rows_hbm.at[pl.ds(dst, 16)])


def _sc_gather(lists, offs, cnts, packed):
    mesh = plsc.VectorSubcoreMesh(core_axis_name="c", subcore_axis_name="s")
    kern = pl.kernel(
        _sc2_body,
        out_type=jax.ShapeDtypeStruct((G, 128), jnp.float32),
        mesh=mesh,
        scratch_types=[
            pltpu.VMEM((LPC,), jnp.int32),
            pltpu.VMEM((16,), jnp.int32),
            pltpu.VMEM((16,), jnp.int32),
            pltpu.VMEM((16,), jnp.int32),
            pltpu.VMEM((16, 128), jnp.float32),
            pltpu.SemaphoreType.DMA,
        ],
        compiler_params=_sc_cp(),
    )
    return kern(lists, offs, cnts, packed)


def _nms_body(rows_ref, cnt_ref, out_ref,
              bits_ref, idx_ref, sorted_ref,
              xmin_ref, ymin_ref, xmax_ref, ymax_ref, area_ref, mask_ref):
    f32 = jnp.float32
    i32 = jnp.int32

    rows = rows_ref[:, 0:16]
    rt = jnp.transpose(rows)

    def col(k):
        return rt[k].reshape(36, 128)

    score = col(0)
    u0 = cnt_ref[0, 0]
    u1 = cnt_ref[0, 16]
    r36 = lax.broadcasted_iota(i32, (36, 128), 0)
    l36 = lax.broadcasted_iota(i32, (36, 128), 1)
    pos = r36 * 128 + l36
    vpos = (pos < u0) | ((pos >= PC) & (pos < PC + u1))
    score = jnp.where(vpos, score, 0.0)
    xsf = jnp.where(vpos, col(6), 0.0)
    ysf = jnp.where(vpos, col(7), 0.0)
    bits = lax.bitcast_convert_type(score, i32)
    idxf = ysf * 320.0 + xsf

    bits_ref[...] = bits.reshape(1, G)
    idx_ref[...] = idxf.reshape(1, G)
    pcol = lax.broadcasted_iota(i32, (G, 1), 0)
    vcol = (pcol < u0) | ((pcol >= PC) & (pcol < PC + u1))
    rows = jnp.where(vcol, rows, 0.0)

    bits_c = bits_ref[...]
    idx_c = idx_ref[...]

    def rank_body(rb, ranks):
        bj = jnp.transpose(bits_ref[0:1, pl.ds(rb * 128, 128)])
        ij = jnp.transpose(idx_ref[0:1, pl.ds(rb * 128, 128)])
        gt = bj > bits_c
        tie = (bj == bits_c) & (ij < idx_c)
        return ranks + jnp.sum((gt | tie).astype(i32), axis=0,
                               keepdims=True)

    ranks = lax.fori_loop(0, G // 128, rank_body,
                          jnp.zeros((1, G), i32))
    realc = bits_c != 0

    def perm_body(sb, carry):
        slots = sb * 256 + lax.broadcasted_iota(i32, (256, 1), 0)
        pmat = ((ranks == slots) & realc).astype(f32)
        blk = jax.lax.dot_general(pmat, rows,
                                  (((1,), (0,)), ((), ())),
                                  precision=jax.lax.Precision.HIGHEST,
                                  preferred_element_type=f32)
        sorted_ref[pl.ds(sb * 256, 256), :] = blk
        return carry

    lax.fori_loop(0, S // 256, perm_body, jnp.int32(0))

    srt = jnp.transpose(sorted_ref[...])

    def scol(k):
        return srt[k].reshape(16, 128)

    sscore = scol(0)
    d0, d1, d2, d3 = scol(1), scol(2), scol(3), scol(4)
    ang = scol(5)
    x0 = scol(6) * 4.0
    y0 = scol(7) * 4.0

    wsum = d1 + d3
    hsum = d0 + d2
    c0 = jnp.cos(ang)
    s0 = jnp.sin(ang)
    p4x = c0 * d3 + s0 * (-d2)
    p4y = -s0 * d3 + c0 * (-d2)
    ox = x0 - p4x
    oy = y0 - p4y
    q0x = s0 * (-hsum) + ox
    q0y = c0 * (-hsum) + oy
    q1x = c0 * wsum + s0 * (-hsum) + ox
    q1y = -s0 * wsum + c0 * (-hsum) + oy
    q2x = c0 * wsum + ox
    q2y = -s0 * wsum + oy
    q3x = ox
    q3y = oy
    c1 = jnp.cos(-ang)
    s1 = jnp.sin(-ang)
    r4x = c1 * (-d1) - s1 * (-d2)
    r4y = s1 * (-d1) + c1 * (-d2)
    ox1 = x0 - r4x
    oy1 = y0 - r4y
    r0x = c1 * (-wsum) - s1 * (-hsum) + ox1
    r0y = s1 * (-wsum) + c1 * (-hsum) + oy1
    r1x = -s1 * (-hsum) + ox1
    r1y = c1 * (-hsum) + oy1
    r2x = ox1
    r2y = oy1
    r3x = c1 * (-wsum) + ox1
    r3y = s1 * (-wsum) + oy1

    ge = ang >= 0.0
    xa = jnp.where(ge, q0x, r0x)
    xb = jnp.where(ge, q1x, r1x)
    xc = jnp.where(ge, q2x, r2x)
    xd = jnp.where(ge, q3x, r3x)
    ya = jnp.where(ge, q0y, r0y)
    yb = jnp.where(ge, q1y, r1y)
    yc = jnp.where(ge, q2y, r2y)
    yd = jnp.where(ge, q3y, r3y)
    xmin = jnp.minimum(jnp.minimum(xa, xb), jnp.minimum(xc, xd))
    xmax = jnp.maximum(jnp.maximum(xa, xb), jnp.maximum(xc, xd))
    ymin = jnp.minimum(jnp.minimum(ya, yb), jnp.minimum(yc, yd))
    ymax = jnp.maximum(jnp.maximum(ya, yb), jnp.maximum(yc, yd))
    area = jnp.maximum(xmax - xmin, 0.0) * jnp.maximum(ymax - ymin, 0.0)

    xmin_ref[...] = xmin
    ymin_ref[...] = ymin
    xmax_ref[...] = xmax
    ymax_ref[...] = ymax
    area_ref[...] = area

    l128c = lax.broadcasted_iota(i32, (1, 128), 1)
    r128 = lax.broadcasted_iota(i32, (128, 1), 0)

    def mrow_body(rb, carry):
        xnr = jnp.transpose(xmin_ref[pl.ds(rb, 1), :])
        ynr = jnp.transpose(ymin_ref[pl.ds(rb, 1), :])
        xxr = jnp.transpose(xmax_ref[pl.ds(rb, 1), :])
        yxr = jnp.transpose(ymax_ref[pl.ds(rb, 1), :])
        ar = jnp.transpose(area_ref[pl.ds(rb, 1), :])
        rslot = rb * 128 + r128
        tiles = []
        for g in range(16):
            xnc = xmin_ref[g:g + 1, :]
            ync = ymin_ref[g:g + 1, :]
            xxc = xmax_ref[g:g + 1, :]
            yxc = ymax_ref[g:g + 1, :]
            ac = area_ref[g:g + 1, :]
            ix1 = jnp.maximum(xnr, xnc)
            iy1 = jnp.maximum(ynr, ync)
            ix2 = jnp.minimum(xxr, xxc)
            iy2 = jnp.minimum(yxr, yxc)
            inter = (jnp.maximum(ix2 - ix1, 0.0)
                     * jnp.maximum(iy2 - iy1, 0.0))
            union = ar + ac - inter
            cslot = g * 128 + l128c
            m = ((inter > NMS_T * union) & (cslot > rslot)).astype(f32)
            tiles.append(m)
        mask_ref[pl.ds(rb * 128, 128), :] = jnp.concatenate(tiles, axis=1)
        return carry

    lax.fori_loop(0, S // 128, mrow_body, jnp.int32(0))

    r16 = lax.broadcasted_iota(i32, (16, 128), 0)
    l16 = lax.broadcasted_iota(i32, (16, 128), 1)
    slotpos = r16 * 128 + l16
    valid = ((sscore > 0.8) & (slotpos < K)).astype(f32)

    keep = valid.reshape(1, S)
    sub8 = lax.broadcasted_iota(i32, (8, 128), 0)
    for b in range(S // 128):
        kb0 = keep[:, b * 128:(b + 1) * 128]

        def ib_body(c, kb, _b=b):
            src = pl.multiple_of(_b * 128 + c * 8, 8)
            tile8 = mask_ref[pl.ds(src, 8),
                             _b * 128:(_b + 1) * 128]
            for r in range(8):
                i = c * 8 + r
                ki = jnp.max(jnp.where(l128c == i, kb, 0.0))
                row = jnp.max(jnp.where(sub8 == r, tile8, 0.0),
                              axis=0, keepdims=True)
                kb = kb * (1.0 - ki * row)
            return kb

        kbf = lax.fori_loop(0, 16, ib_body, kb0)
        spread = jax.lax.dot_general(
            kbf, mask_ref[pl.ds(b * 128, 128), :],
            (((1,), (0,)), ((), ())), preferred_element_type=f32)
        keep = keep * (1.0 - (spread > 0.0).astype(f32))
    keep = keep.reshape(16, 128)

    b0 = xmin * keep
    b1 = ymin * keep
    b2 = xmax * keep
    b3 = ymax * keep
    sc = sscore * keep
    five = jnp.stack([b0, b1, b2, b3, sc], axis=0)
    out_ref[...] = jnp.transpose(five.reshape(5, S))[:K, :]


def _nms(rows, cpad):
    return pl.pallas_call(
        _nms_body,
        out_shape=jax.ShapeDtypeStruct((K, 5), jnp.float32),
        scratch_shapes=[
            pltpu.VMEM((1, G), jnp.int32),
            pltpu.VMEM((1, G), jnp.float32),
            pltpu.VMEM((S, 16), jnp.float32),
            pltpu.VMEM((16, 128), jnp.float32),
            pltpu.VMEM((16, 128), jnp.float32),
            pltpu.VMEM((16, 128), jnp.float32),
            pltpu.VMEM((16, 128), jnp.float32),
            pltpu.VMEM((16, 128), jnp.float32),
            pltpu.VMEM((S, S), jnp.float32),
        ],
    )(rows, cpad)


def kernel(score_map, geo_map):
    score_flat = score_map.reshape(-1)
    thr = _thresholds(score_flat.reshape(800, 128)).reshape(-1)
    idx = jnp.arange(N, dtype=jnp.int32)
    xs = (idx % W).astype(jnp.float32)
    ys = (idx // W).astype(jnp.float32)
    packed = jnp.concatenate(
        [score_flat[:, None], geo_map.reshape(N, 5),
         xs[:, None], ys[:, None]], axis=1)
    packed = jnp.pad(packed, ((0, NPAD - N), (0, 120)))
    lists, wcnt = _sc_compact(score_flat, thr)
    offs, cnts, used = _prefix(wcnt.reshape(32, 16))
    rows = _sc_gather(lists, offs.reshape(512), cnts.reshape(512), packed)
    cpad = jnp.pad(used.reshape(32), (0, 96)).reshape(1, 128)
    return _nms(rows, cpad)
